# baseline (speedup 1.0000x reference)
import jax
import jax.numpy as jnp
from jax import lax
from jax.experimental import pallas as pl
from jax.experimental.pallas import tpu as pltpu

H, W = 32, 32
PER = H * W
NMAT = 3
N = NMAT * PER
LANE = 8
NOPS = PER
B = 4
TB = 256


def _f32(x):
    return x.astype(jnp.float32)


def _transpose_row(row):
    ones11 = jnp.ones((1, 1), jnp.float32)
    return lax.dot_general(row, ones11, (((0,), (0,)), ((), ())),
                           preferred_element_type=jnp.float32,
                           precision=lax.Precision.HIGHEST)


def _shift_minor(x, k):
    if k == 0:
        return x
    r = x.shape[0]
    z = jnp.zeros((r, abs(k)), x.dtype)
    if k > 0:
        return jnp.concatenate([x[:, k:], z], axis=1)
    return jnp.concatenate([z, x[:, :k]], axis=1)


def _conv_patches(x, width):
    c_count, p = x.shape
    col = lax.broadcasted_iota(jnp.int32, (c_count, p), 1) % width
    rows = []
    for ky in range(3):
        for kx in range(3):
            k = (ky - 1) * width + (kx - 1)
            sh = _shift_minor(x, k)
            if kx == 0:
                sh = jnp.where(col >= 1, sh, 0.0)
            elif kx == 2:
                sh = jnp.where(col <= width - 2, sh, 0.0)
            rows.append(sh)
    return jnp.concatenate(rows, axis=0)


def _sort_key_halves(x_row):
    bits = lax.bitcast_convert_type(x_row + 0.0, jnp.int32)
    sign = lax.shift_right_arithmetic(bits, 31)
    key = bits ^ (sign | jnp.int32(-2 ** 31))
    hi = _f32(lax.shift_right_logical(key, 16))
    lo = _f32(key & jnp.int32(0xFFFF))
    return hi, lo


def _rank_row(hi_row, lo_row, hi_col, lo_col, n):
    acc = jnp.zeros((1, n), jnp.float32)
    iidx = lax.broadcasted_iota(jnp.int32, (TB, n), 1)
    jbase = lax.broadcasted_iota(jnp.int32, (TB, n), 0)
    for bj in range(n // TB):
        hcb = hi_col[bj * TB:(bj + 1) * TB, :]
        lcb = lo_col[bj * TB:(bj + 1) * TB, :]
        jidx = jbase + (bj * TB)
        cmp = ((hcb < hi_row)
               | ((hcb == hi_row)
                  & ((lcb < lo_row)
                     | ((lcb == lo_row) & (jidx < iidx)))))
        acc = acc + jnp.sum(_f32(cmp), axis=0, keepdims=True)
    return acc


def _pl_terms(logits_row, rank_col, n):
    m = jnp.max(logits_row)
    e_row = jnp.exp(logits_row - m)
    jidx = _f32(lax.broadcasted_iota(jnp.int32, (TB, n), 1))
    r_acc = jnp.zeros((1, n), jnp.float32)
    for bi in range(n // TB):
        rcb = rank_col[bi * TB:(bi + 1) * TB, :]
        mge = _f32(rcb >= jidx)
        e_blk = e_row[:, bi * TB:(bi + 1) * TB]
        r_acc = r_acc + jnp.dot(e_blk, mge,
                                preferred_element_type=jnp.float32, precision=lax.Precision.HIGHEST)
    return jnp.sum(logits_row) - jnp.sum(jnp.log(r_acc)) - n * m


def _body(ml_ref, gm_ref, gop_ref, wa_ref, ba_ref, w2_ref, b2_ref,
          mp_ref, pw_ref, pb_ref, o_ref):
    ml_row = ml_ref[...].reshape(1, N)
    g_row = gm_ref[...].reshape(1, N)
    u_row = ml_row + g_row
    hi_row, lo_row = _sort_key_halves(u_row)
    hi_col = _transpose_row(hi_row)
    lo_col = _transpose_row(lo_row)

    rank_row = _rank_row(hi_row, lo_row, hi_col, lo_col, N)
    rank_col = _transpose_row(rank_row)
    mem_lp = _pl_terms(ml_row, rank_col, N)

    jidx_f = _f32(lax.broadcasted_iota(jnp.int32, (TB, N), 1))
    ivals = _f32(lax.broadcasted_iota(jnp.int32, (1, N), 1))
    perm_acc = jnp.zeros((1, N), jnp.float32)
    for bi in range(N // TB):
        rcb = rank_col[bi * TB:(bi + 1) * TB, :]
        meq = _f32(rcb == jidx_f)
        iv = ivals[:, bi * TB:(bi + 1) * TB]
        perm_acc = perm_acc + jnp.dot(iv, meq,
                                      preferred_element_type=jnp.float32, precision=lax.Precision.HIGHEST)
    perm_row = perm_acc

    feats = []
    for mm in range(NMAT):
        pslice = perm_row[:, mm * PER:(mm + 1) * PER]
        patches = _conv_patches(pslice, W)
        wa = wa_ref[mm]
        ba = ba_ref[mm]
        feats.append(jax.nn.relu(
            jnp.dot(wa, patches, preferred_element_type=jnp.float32) + ba))
    feat = jnp.concatenate(feats, axis=1)

    jidx0 = _f32(lax.broadcasted_iota(jnp.int32, (N, TB), 0))
    mem_cols = []
    for bi in range(N // TB):
        rrb = rank_row[:, bi * TB:(bi + 1) * TB]
        p2 = _f32(jidx0 == rrb)
        mem_cols.append(jnp.dot(feat, p2,
                                preferred_element_type=jnp.float32, precision=lax.Precision.HIGHEST))
    mem_flat = jnp.concatenate(mem_cols, axis=1)

    x2 = _conv_patches(mem_flat, LANE)
    mc = jax.nn.relu(jnp.dot(w2_ref[...], x2,
                             preferred_element_type=jnp.float32) + b2_ref[...])
    pooled = jnp.dot(mc, mp_ref[...],
                     preferred_element_type=jnp.float32, precision=lax.Precision.HIGHEST)
    op_row = pb_ref[...]
    for c in range(16):
        op_row = op_row + jnp.dot(pooled[c:c + 1, :], pw_ref[c],
                                  preferred_element_type=jnp.float32)

    u2_row = op_row + gop_ref[...].reshape(1, NOPS)
    hi2_row, lo2_row = _sort_key_halves(u2_row)
    hi2_col = _transpose_row(hi2_row)
    lo2_col = _transpose_row(lo2_row)
    rank2_row = _rank_row(hi2_row, lo2_row, hi2_col, lo2_col, NOPS)
    rank2_col = _transpose_row(rank2_row)
    op_lp = _pl_terms(op_row, rank2_col, NOPS)

    ps = jnp.concatenate(
        [perm_row[:, r * PER:(r + 1) * PER] for r in range(3)], axis=0)
    tidx = _f32(lax.broadcasted_iota(jnp.int32, (1, TB), 1))
    s_cols = []
    for bt in range(NOPS // TB):
        qt = _f32(rank2_col == (tidx + bt * TB))
        s_cols.append(jnp.dot(ps, qt, preferred_element_type=jnp.float32, precision=lax.Precision.HIGHEST))
    s_mat = jnp.concatenate(s_cols, axis=1)

    d1 = s_mat[1:2, :] - s_mat[0:1, :]
    d2 = s_mat[2:3, :] - s_mat[1:2, :]
    intra = (jnp.sum(jax.nn.relu(d1)) + jnp.sum(jax.nn.relu(d2))
             + jnp.sum(jax.nn.relu(-d1) ** 2) + jnp.sum(jax.nn.relu(-d2) ** 2))
    di = s_mat[0:1, 1:] - s_mat[2:3, :-1]
    inter = jnp.sum(jax.nn.relu(di)) + jnp.sum(jax.nn.relu(-di) ** 2)

    lane_i = lax.broadcasted_iota(jnp.int32, (1, 128), 1)
    vec = jnp.where(lane_i == 0, inter,
                    jnp.where(lane_i == 1, intra,
                              jnp.where(lane_i == 2, op_lp, mem_lp)))
    o_ref[...] = vec.reshape(1, 1, 128)


def kernel(mem_logits_batch, convA_w0, convA_b0, convA_w1, convA_b1,
           convA_w2, convA_b2, conv2_w, conv2_b, proj_w, proj_b):
    f32 = jnp.float32
    gms, gops = [], []
    for b in range(B):
        key = jax.random.fold_in(jax.random.key(42), b)
        kg1, kg2 = jax.random.split(key)
        gms.append(-jnp.log(jax.random.exponential(kg1, (N,)) + 1e-20))
        gops.append(-jnp.log(jax.random.exponential(kg2, (NOPS,)) + 1e-20))
    gm = jnp.stack(gms).reshape(B, 1, N).astype(f32)
    gop = jnp.stack(gops).reshape(B, 1, NOPS).astype(f32)

    ml = mem_logits_batch.reshape(B, 1, N).astype(f32)
    wa = jnp.stack([convA_w0.reshape(8, 9), convA_w1.reshape(8, 9),
                    convA_w2.reshape(8, 9)]).astype(f32)
    ba = jnp.stack([convA_b0, convA_b1, convA_b2]).reshape(3, 8, 1).astype(f32)
    w2m = conv2_w.transpose(0, 2, 3, 1).reshape(16, 72).astype(f32)
    b2c = conv2_b.reshape(16, 1).astype(f32)
    i = jnp.arange(N)
    r, c = i // LANE, i % LANE
    sel = (r // 96) * 4 + (c // 2)
    mpool = (jax.nn.one_hot(sel, 16, dtype=f32) / 192.0)
    pw3 = proj_w.T.reshape(16, 16, NOPS).astype(f32)
    pbr = proj_b.reshape(1, NOPS).astype(f32)

    out = pl.pallas_call(
        _body,
        grid=(B,),
        in_specs=[
            pl.BlockSpec((1, 1, N), lambda b: (b, 0, 0)),
            pl.BlockSpec((1, 1, N), lambda b: (b, 0, 0)),
            pl.BlockSpec((1, 1, NOPS), lambda b: (b, 0, 0)),
            pl.BlockSpec((3, 8, 9), lambda b: (0, 0, 0)),
            pl.BlockSpec((3, 8, 1), lambda b: (0, 0, 0)),
            pl.BlockSpec((16, 72), lambda b: (0, 0)),
            pl.BlockSpec((16, 1), lambda b: (0, 0)),
            pl.BlockSpec((N, 16), lambda b: (0, 0)),
            pl.BlockSpec((16, 16, NOPS), lambda b: (0, 0, 0)),
            pl.BlockSpec((1, NOPS), lambda b: (0, 0)),
        ],
        out_specs=pl.BlockSpec((1, 1, 128), lambda b: (b, 0, 0)),
        out_shape=jax.ShapeDtypeStruct((B, 1, 128), f32),
    )(ml, gm, gop, wa, ba, w2m, b2c, mpool, pw3, pbr)

    res = out[:, 0, :]
    return res[:, 0], res[:, 1], res[:, 2], res[:, 3]


# speedup vs baseline: 1.1691x; 1.1691x over previous
import jax
import jax.numpy as jnp
from jax import lax
from jax.experimental import pallas as pl

H, W = 32, 32
PER = H * W
NMAT = 3
N = NMAT * PER
LANE = 8
NOPS = PER
B = 4
TB = 256
MINI = -2 ** 31


def _f32(x):
    return x.astype(jnp.float32)


def _hp(a, b):
    return jnp.dot(a, b, preferred_element_type=jnp.float32,
                   precision=lax.Precision.HIGHEST)


def _transpose_row(row):
    ones11 = jnp.ones((1, 1), jnp.float32)
    return lax.dot_general(row, ones11, (((0,), (0,)), ((), ())),
                           preferred_element_type=jnp.float32,
                           precision=lax.Precision.HIGHEST)


def _shift_minor(x, k):
    if k == 0:
        return x
    r = x.shape[0]
    z = jnp.zeros((r, abs(k)), x.dtype)
    if k > 0:
        return jnp.concatenate([x[:, k:], z], axis=1)
    return jnp.concatenate([z, x[:, :k]], axis=1)


def _suffix_sums(row, n):
    acc = row
    sh = 1
    while sh < n:
        acc = acc + _shift_minor(acc, sh)
        sh *= 2
    return acc


def _conv_patches(x, width):
    c_count, p = x.shape
    col = lax.broadcasted_iota(jnp.int32, (c_count, p), 1) % width
    rows = []
    for ky in range(3):
        for kx in range(3):
            k = (ky - 1) * width + (kx - 1)
            sh = _shift_minor(x, k)
            if kx == 0:
                sh = jnp.where(col >= 1, sh, 0.0)
            elif kx == 2:
                sh = jnp.where(col <= width - 2, sh, 0.0)
            rows.append(sh)
    return jnp.concatenate(rows, axis=0)


def _sort_keys(x_row):
    bits = lax.bitcast_convert_type(x_row + 0.0, jnp.int32)
    sign = lax.shift_right_arithmetic(bits, 31)
    key_u = bits ^ (sign | jnp.int32(MINI))
    key_s = key_u ^ jnp.int32(MINI)
    hi = _f32(lax.shift_right_logical(key_u, 16))
    lo = _f32(key_u & jnp.int32(0xFFFF))
    return key_s, hi, lo


def _key_col(hi_col, lo_col):
    h = hi_col.astype(jnp.int32) ^ jnp.int32(32768)
    return lax.shift_left(h, 16) | lo_col.astype(jnp.int32)


def _rank_row(key_row, key_col, n):
    acc = jnp.zeros((1, n), jnp.float32)
    iidx = lax.broadcasted_iota(jnp.int32, (TB, n), 1)
    jbase = lax.broadcasted_iota(jnp.int32, (TB, n), 0)
    d = iidx - jbase
    for bj in range(n // TB):
        kcb = key_col[bj * TB:(bj + 1) * TB, :]
        cmp = (kcb < key_row) | ((kcb == key_row) & (d > bj * TB))
        acc = acc + jnp.sum(_f32(cmp), axis=0, keepdims=True)
    return acc


def _ranks(x_row, n):
    key_row, hi, lo = _sort_keys(x_row)
    key_col = _key_col(_transpose_row(hi), _transpose_row(lo))
    rank_row = _rank_row(key_row, key_col, n)
    rank_col = _transpose_row(rank_row)
    return rank_row, rank_col


def _body(ml_ref, gm_ref, gop_ref, wa_ref, ba_ref, w2_ref, b2_ref,
          mp_ref, pw_ref, pb_ref, o_ref):
    ml_row = ml_ref[...].reshape(1, N)
    g_row = gm_ref[...].reshape(1, N)
    u_row = ml_row + g_row
    rank_row, rank_col = _ranks(u_row, N)

    m1 = jnp.max(ml_row)
    e_row = jnp.exp(ml_row - m1)

    jidx_f = _f32(lax.broadcasted_iota(jnp.int32, (TB, N), 1))
    ivals = _f32(lax.broadcasted_iota(jnp.int32, (1, N), 1))
    perm_acc = jnp.zeros((1, N), jnp.float32)
    es_acc = jnp.zeros((1, N), jnp.float32)
    for bi in range(N // TB):
        rcb = rank_col[bi * TB:(bi + 1) * TB, :]
        meq = _f32(rcb == jidx_f)
        sl = slice(bi * TB, (bi + 1) * TB)
        perm_acc = perm_acc + _hp(ivals[:, sl], meq)
        es_acc = es_acc + _hp(e_row[:, sl], meq)
    perm_row = perm_acc

    rsuf = _suffix_sums(es_acc, N)
    mem_lp = jnp.sum(ml_row) - jnp.sum(jnp.log(rsuf)) - N * m1

    feats = []
    for mm in range(NMAT):
        patches = _conv_patches(perm_row[:, mm * PER:(mm + 1) * PER], W)
        feats.append(jax.nn.relu(
            jnp.dot(wa_ref[mm], patches,
                    preferred_element_type=jnp.float32) + ba_ref[mm]))
    feat = jnp.concatenate(feats, axis=1)

    jidx0 = lax.broadcasted_iota(jnp.int32, (N, TB), 0)
    mem_cols = []
    for bi in range(N // TB):
        rrb = rank_row[:, bi * TB:(bi + 1) * TB].astype(jnp.int32)
        p2 = _f32(jidx0 == rrb)
        mem_cols.append(_hp(feat, p2))
    mem_flat = jnp.concatenate(mem_cols, axis=1)

    x2 = _conv_patches(mem_flat, LANE)
    mc = jax.nn.relu(jnp.dot(w2_ref[...], x2,
                             preferred_element_type=jnp.float32)
                     + b2_ref[...])
    pooled = _hp(mc, mp_ref[...])
    op_row = pb_ref[...]
    for c in range(16):
        op_row = op_row + jnp.dot(pooled[c:c + 1, :], pw_ref[c],
                                  preferred_element_type=jnp.float32)

    u2_row = op_row + gop_ref[...].reshape(1, NOPS)
    _, rank2_col = _ranks(u2_row, NOPS)

    m2 = jnp.max(op_row)
    e2_row = jnp.exp(op_row - m2)
    ps = jnp.concatenate(
        [perm_row[:, r * PER:(r + 1) * PER] for r in range(3)], axis=0)
    pe = jnp.concatenate([ps, e2_row], axis=0)
    tidx = _f32(lax.broadcasted_iota(jnp.int32, (NOPS, TB), 1))
    s_cols = []
    for bt in range(NOPS // TB):
        qt = _f32(rank2_col == (tidx + bt * TB))
        s_cols.append(_hp(pe, qt))
    se = jnp.concatenate(s_cols, axis=1)
    s_mat = se[0:3, :]

    r2suf = _suffix_sums(se[3:4, :], NOPS)
    op_lp = jnp.sum(op_row) - jnp.sum(jnp.log(r2suf)) - NOPS * m2

    d1 = s_mat[1:2, :] - s_mat[0:1, :]
    d2 = s_mat[2:3, :] - s_mat[1:2, :]
    intra = (jnp.sum(jax.nn.relu(d1)) + jnp.sum(jax.nn.relu(d2))
             + jnp.sum(jax.nn.relu(-d1) ** 2)
             + jnp.sum(jax.nn.relu(-d2) ** 2))
    di = s_mat[0:1, 1:] - s_mat[2:3, :-1]
    inter = jnp.sum(jax.nn.relu(di)) + jnp.sum(jax.nn.relu(-di) ** 2)

    lane_i = lax.broadcasted_iota(jnp.int32, (1, 128), 1)
    vec = jnp.where(lane_i == 0, inter,
                    jnp.where(lane_i == 1, intra,
                              jnp.where(lane_i == 2, op_lp, mem_lp)))
    o_ref[...] = vec.reshape(1, 1, 128)


def kernel(mem_logits_batch, convA_w0, convA_b0, convA_w1, convA_b1,
           convA_w2, convA_b2, conv2_w, conv2_b, proj_w, proj_b):
    f32 = jnp.float32
    gms, gops = [], []
    for b in range(B):
        key = jax.random.fold_in(jax.random.key(42), b)
        kg1, kg2 = jax.random.split(key)
        gms.append(-jnp.log(jax.random.exponential(kg1, (N,)) + 1e-20))
        gops.append(-jnp.log(jax.random.exponential(kg2, (NOPS,)) + 1e-20))
    gm = jnp.stack(gms).reshape(B, 1, N).astype(f32)
    gop = jnp.stack(gops).reshape(B, 1, NOPS).astype(f32)

    ml = mem_logits_batch.reshape(B, 1, N).astype(f32)
    wa = jnp.stack([convA_w0.reshape(8, 9), convA_w1.reshape(8, 9),
                    convA_w2.reshape(8, 9)]).astype(f32)
    ba = jnp.stack([convA_b0, convA_b1, convA_b2]).reshape(3, 8, 1).astype(f32)
    w2m = conv2_w.transpose(0, 2, 3, 1).reshape(16, 72).astype(f32)
    b2c = conv2_b.reshape(16, 1).astype(f32)
    i = jnp.arange(N)
    sel = (i // LANE // 96) * 4 + ((i % LANE) // 2)
    mpool = (jax.nn.one_hot(sel, 16, dtype=f32) / 192.0)
    pw3 = proj_w.T.reshape(16, 16, NOPS).astype(f32)
    pbr = proj_b.reshape(1, NOPS).astype(f32)

    out = pl.pallas_call(
        _body,
        grid=(B,),
        in_specs=[
            pl.BlockSpec((1, 1, N), lambda b: (b, 0, 0)),
            pl.BlockSpec((1, 1, N), lambda b: (b, 0, 0)),
            pl.BlockSpec((1, 1, NOPS), lambda b: (b, 0, 0)),
            pl.BlockSpec((3, 8, 9), lambda b: (0, 0, 0)),
            pl.BlockSpec((3, 8, 1), lambda b: (0, 0, 0)),
            pl.BlockSpec((16, 72), lambda b: (0, 0)),
            pl.BlockSpec((16, 1), lambda b: (0, 0)),
            pl.BlockSpec((N, 16), lambda b: (0, 0)),
            pl.BlockSpec((16, 16, NOPS), lambda b: (0, 0, 0)),
            pl.BlockSpec((1, NOPS), lambda b: (0, 0)),
        ],
        out_specs=pl.BlockSpec((1, 1, 128), lambda b: (b, 0, 0)),
        out_shape=jax.ShapeDtypeStruct((B, 1, 128), f32),
    )(ml, gm, gop, wa, ba, w2m, b2c, mpool, pw3, pbr)

    res = out[:, 0, :]
    return res[:, 0], res[:, 1], res[:, 2], res[:, 3]
